# baseline (speedup 1.0000x reference)
import functools
import math

import jax
import jax.numpy as jnp
from jax.experimental import pallas as pl

H = 16


def _prep_kernel(wq_ref, wk_ref, wv_ref, oq_ref, ok_ref, ov_ref, *, scale):
    oq_ref[...] = (wq_ref[...] * scale).astype(jnp.bfloat16)
    ok_ref[...] = wk_ref[...].astype(jnp.bfloat16)
    ov_ref[...] = wv_ref[...].astype(jnp.bfloat16)


def _prep(Wq, Wk, Wv, scale, g=16):
    K, N = Wq.shape
    blk = pl.BlockSpec((K // g, N), lambda i: (i, 0))
    out = jax.ShapeDtypeStruct((K, N), jnp.bfloat16)
    return pl.pallas_call(
        functools.partial(_prep_kernel, scale=scale),
        grid=(g,),
        in_specs=[blk, blk, blk],
        out_specs=[blk, blk, blk],
        out_shape=[out, out, out],
    )(Wq, Wk, Wv)


def _qkv_kernel(x1_ref, x2_ref, x3_ref, w1_ref, w2_ref, w3_ref,
                b1_ref, b2_ref, b3_ref, wo_ref,
                o1_ref, o2_ref, o3_ref, wob_ref):
    for x_ref, w_ref, b_ref, o_ref in (
        (x1_ref, w1_ref, b1_ref, o1_ref),
        (x2_ref, w2_ref, b2_ref, o2_ref),
        (x3_ref, w3_ref, b3_ref, o3_ref),
    ):
        xb = x_ref[...].astype(jnp.bfloat16)
        acc = jnp.dot(xb, w_ref[...], preferred_element_type=jnp.float32)
        o_ref[...] = (acc + b_ref[...]).astype(jnp.bfloat16)
    wob_ref[...] = wo_ref[...].astype(jnp.bfloat16)


def _qkv_proj(q2, k2, v2, Wqs, bqs, Wk, bk, Wv, bv, Wo, bm=256):
    M, K = q2.shape
    N = Wqs.shape[1]
    g = M // bm
    ws = N // g
    bf = jnp.bfloat16
    blk_x = pl.BlockSpec((bm, K), lambda i: (i, 0))
    blk_w = pl.BlockSpec((K, N), lambda i: (0, 0))
    blk_b = pl.BlockSpec((1, N), lambda i: (0, 0))
    blk_ws = pl.BlockSpec((ws, N), lambda i: (i, 0))
    blk_o = pl.BlockSpec((bm, N), lambda i: (i, 0))
    out = jax.ShapeDtypeStruct((M, N), bf)
    return pl.pallas_call(
        _qkv_kernel,
        grid=(g,),
        in_specs=[blk_x, blk_x, blk_x, blk_w, blk_w, blk_w,
                  blk_b, blk_b, blk_b, blk_ws],
        out_specs=[blk_o, blk_o, blk_o, blk_ws],
        out_shape=[out, out, out, jax.ShapeDtypeStruct((K, N), bf)],
    )(q2, k2, v2, Wqs, Wk, Wv,
      bqs.reshape(1, N), bk.reshape(1, N), bv.reshape(1, N), Wo)


def _attn_out_kernel(q_ref, k_ref, v_ref, wo_ref, bo_ref, o_ref, ctx_ref):
    D = q_ref.shape[1]
    S = k_ref.shape[0]
    DK = D // H
    C = 256
    nc = S // C
    tokens = []
    for h in range(H):
        lo, hi = h * DK, (h + 1) * DK
        qh = q_ref[:, lo:hi]
        if h >= 2:
            qh = (qh.astype(jnp.float32)
                  + jnp.minimum(tokens[h - 2], 0.0)).astype(jnp.bfloat16)
        m = l = acc = None
        for j in range(nc):
            kj = k_ref[j * C:(j + 1) * C, lo:hi]
            s = jax.lax.dot_general(
                qh, kj, (((1,), (1,)), ((), ())),
                preferred_element_type=jnp.float32,
            )
            mj = jnp.max(s, axis=-1, keepdims=True)
            mnew = mj if m is None else jnp.maximum(m, mj)
            p = jnp.exp2(s - mnew)
            lj = jnp.sum(p, axis=-1, keepdims=True)
            vj = v_ref[j * C:(j + 1) * C, lo:hi]
            dj = jnp.dot(p.astype(jnp.bfloat16), vj,
                         preferred_element_type=jnp.float32)
            if m is None:
                l, acc = lj, dj
            else:
                alpha = jnp.exp2(m - mnew)
                l = l * alpha + lj
                acc = acc * alpha + dj
            m = mnew
        ctx_ref[:, lo:hi] = (acc / l).astype(jnp.bfloat16)
        tokens.append(l)
    o_ref[...] = (
        jnp.dot(ctx_ref[...], wo_ref[...], preferred_element_type=jnp.float32)
        + bo_ref[...]
    )


def _attn_out(qp, kp, vp, Wob, bo, B=2, bq=512):
    from jax.experimental.pallas import tpu as pltpu

    BS, D = qp.shape
    S = BS // B
    nq = S // bq
    outs = []
    for b in range(B):
        outs.append(pl.pallas_call(
            _attn_out_kernel,
            grid=(nq,),
            in_specs=[
                pl.BlockSpec((bq, D), lambda i, b=b: (b * nq + i, 0)),
                pl.BlockSpec((S, D), lambda i, b=b: (b, 0)),
                pl.BlockSpec((S, D), lambda i, b=b: (b, 0)),
                pl.BlockSpec((D, D), lambda i: (0, 0)),
                pl.BlockSpec((1, D), lambda i: (0, 0)),
            ],
            out_specs=pl.BlockSpec((bq, D), lambda i: (i, 0)),
            out_shape=jax.ShapeDtypeStruct((S, D), jnp.float32),
            scratch_shapes=[pltpu.VMEM((bq, D), jnp.bfloat16)],
        )(qp, kp, vp, Wob, bo.reshape(1, D)))
    return jnp.stack(outs)


@jax.jit
def kernel(query, key, value, Wq, bq, Wk, bk, Wv, bv, Wo, bo):
    B, S, D = query.shape
    bf = jnp.bfloat16
    scale = math.log2(math.e) / math.sqrt(D // H)

    Wqs, Wkb, Wvb = _prep(Wq, Wk, Wv, scale)
    qp, kp, vp, Wob = _qkv_proj(
        query.reshape(B * S, D),
        key.reshape(B * S, D),
        value.reshape(B * S, D),
        Wqs, bq * scale,
        Wkb, bk,
        Wvb, bv,
        Wo,
    )

    out = _attn_out(qp, kp, vp, Wob, bo, B=B)
    return out


# speedup vs baseline: 1.0056x; 1.0056x over previous
import functools
import math

import jax
import jax.numpy as jnp
from jax.experimental import pallas as pl

H = 16


def _prep_kernel(wq_ref, wk_ref, wv_ref, oq_ref, ok_ref, ov_ref, *, scale):
    oq_ref[...] = (wq_ref[...] * scale).astype(jnp.bfloat16)
    ok_ref[...] = wk_ref[...].astype(jnp.bfloat16)
    ov_ref[...] = wv_ref[...].astype(jnp.bfloat16)


def _prep(Wq, Wk, Wv, scale, g=16):
    K, N = Wq.shape
    blk = pl.BlockSpec((K // g, N), lambda i: (i, 0))
    out = jax.ShapeDtypeStruct((K, N), jnp.bfloat16)
    return pl.pallas_call(
        functools.partial(_prep_kernel, scale=scale),
        grid=(g,),
        in_specs=[blk, blk, blk],
        out_specs=[blk, blk, blk],
        out_shape=[out, out, out],
    )(Wq, Wk, Wv)


def _qkv_kernel(x1_ref, x2_ref, x3_ref, w1_ref, w2_ref, w3_ref,
                b1_ref, b2_ref, b3_ref, wo_ref,
                o1_ref, o2_ref, o3_ref, wob_ref):
    for x_ref, w_ref, b_ref, o_ref in (
        (x1_ref, w1_ref, b1_ref, o1_ref),
        (x2_ref, w2_ref, b2_ref, o2_ref),
        (x3_ref, w3_ref, b3_ref, o3_ref),
    ):
        xb = x_ref[...].astype(jnp.bfloat16)
        acc = jnp.dot(xb, w_ref[...], preferred_element_type=jnp.float32)
        o_ref[...] = (acc + b_ref[...]).astype(jnp.bfloat16)
    wob_ref[...] = wo_ref[...].astype(jnp.bfloat16)


def _qkv_proj(q2, k2, v2, Wqs, bqs, Wk, bk, Wv, bv, Wo, bm=256):
    M, K = q2.shape
    N = Wqs.shape[1]
    g = M // bm
    ws = N // g
    bf = jnp.bfloat16
    blk_x = pl.BlockSpec((bm, K), lambda i: (i, 0))
    blk_w = pl.BlockSpec((K, N), lambda i: (0, 0))
    blk_b = pl.BlockSpec((1, N), lambda i: (0, 0))
    blk_ws = pl.BlockSpec((ws, N), lambda i: (i, 0))
    blk_o = pl.BlockSpec((bm, N), lambda i: (i, 0))
    out = jax.ShapeDtypeStruct((M, N), bf)
    return pl.pallas_call(
        _qkv_kernel,
        grid=(g,),
        in_specs=[blk_x, blk_x, blk_x, blk_w, blk_w, blk_w,
                  blk_b, blk_b, blk_b, blk_ws],
        out_specs=[blk_o, blk_o, blk_o, blk_ws],
        out_shape=[out, out, out, jax.ShapeDtypeStruct((K, N), bf)],
    )(q2, k2, v2, Wqs, Wk, Wv,
      bqs.reshape(1, N), bk.reshape(1, N), bv.reshape(1, N), Wo)


def _attn_out_kernel(q_ref, k_ref, v_ref, wo_ref, bo_ref, o_ref, ctx_ref):
    D = q_ref.shape[1]
    S = k_ref.shape[0]
    DK = D // H
    C = 256
    nc = S // C
    tokens = []
    for h in range(H):
        lo, hi = h * DK, (h + 1) * DK
        qh = q_ref[:, lo:hi]
        if h >= 2:
            qh = (qh.astype(jnp.float32)
                  + jnp.minimum(tokens[h - 2], 0.0)).astype(jnp.bfloat16)
        m = l = acc = None
        for j in range(nc):
            kj = k_ref[j * C:(j + 1) * C, lo:hi]
            s = jax.lax.dot_general(
                qh, kj, (((1,), (1,)), ((), ())),
                preferred_element_type=jnp.float32,
            )
            mj = jnp.max(s, axis=-1, keepdims=True)
            mnew = mj if m is None else jnp.maximum(m, mj)
            p = jnp.exp2(s - mnew)
            lj = jnp.sum(p, axis=-1, keepdims=True)
            vj = v_ref[j * C:(j + 1) * C, lo:hi]
            dj = jnp.dot(p.astype(jnp.bfloat16), vj,
                         preferred_element_type=jnp.float32)
            if m is None:
                l, acc = lj, dj
            else:
                alpha = jnp.exp2(m - mnew)
                l = l * alpha + lj
                acc = acc * alpha + dj
            m = mnew
        ctx_ref[:, lo:hi] = (acc / l).astype(jnp.bfloat16)
        tokens.append(l)
    o_ref[...] = (
        jnp.dot(ctx_ref[...], wo_ref[...], preferred_element_type=jnp.float32)
        + bo_ref[...]
    )


def _attn_out(qp, kp, vp, Wob, bo, B=2, bq=512):
    from jax.experimental.pallas import tpu as pltpu

    BS, D = qp.shape
    S = BS // B
    nq = S // bq
    outs = []
    for b in range(B):
        outs.append(pl.pallas_call(
            _attn_out_kernel,
            grid=(nq,),
            in_specs=[
                pl.BlockSpec((bq, D), lambda i, b=b: (b * nq + i, 0)),
                pl.BlockSpec((S, D), lambda i, b=b: (b, 0)),
                pl.BlockSpec((S, D), lambda i, b=b: (b, 0)),
                pl.BlockSpec((D, D), lambda i: (0, 0)),
                pl.BlockSpec((1, D), lambda i: (0, 0)),
            ],
            out_specs=pl.BlockSpec((bq, D), lambda i: (i, 0)),
            out_shape=jax.ShapeDtypeStruct((S, D), jnp.float32),
            scratch_shapes=[pltpu.VMEM((bq, D), jnp.bfloat16)],
        )(qp, kp, vp, Wob, bo.reshape(1, D)))
    return jnp.stack(outs)


@jax.jit
def kernel(query, key, value, Wq, bq, Wk, bk, Wv, bv, Wo, bo):
    B, S, D = query.shape
    bf = jnp.bfloat16
    scale = math.log2(math.e) / math.sqrt(D // H)

    Wqs = (Wq * scale).astype(bf)
    Wkb = Wk.astype(bf)
    Wvb = Wv.astype(bf)
    qp, kp, vp, Wob = _qkv_proj(
        query.reshape(B * S, D),
        key.reshape(B * S, D),
        value.reshape(B * S, D),
        Wqs, bq * scale,
        Wkb, bk,
        Wvb, bv,
        Wo,
    )

    out = _attn_out(qp, kp, vp, Wob, bo, B=B)
    return out


# speedup vs baseline: 1.0064x; 1.0008x over previous
import functools
import math

import jax
import jax.numpy as jnp
from jax.experimental import pallas as pl

H = 16


def _prep_kernel(wq_ref, wk_ref, wv_ref, oq_ref, ok_ref, ov_ref, *, scale):
    oq_ref[...] = (wq_ref[...] * scale).astype(jnp.bfloat16)
    ok_ref[...] = wk_ref[...].astype(jnp.bfloat16)
    ov_ref[...] = wv_ref[...].astype(jnp.bfloat16)


def _prep(Wq, Wk, Wv, scale, g=16):
    K, N = Wq.shape
    blk = pl.BlockSpec((K // g, N), lambda i: (i, 0))
    out = jax.ShapeDtypeStruct((K, N), jnp.bfloat16)
    return pl.pallas_call(
        functools.partial(_prep_kernel, scale=scale),
        grid=(g,),
        in_specs=[blk, blk, blk],
        out_specs=[blk, blk, blk],
        out_shape=[out, out, out],
    )(Wq, Wk, Wv)


def _qkv_kernel(x1_ref, x2_ref, x3_ref, w1_ref, w2_ref, w3_ref,
                b1_ref, b2_ref, b3_ref, wo_ref,
                o1_ref, o2_ref, o3_ref, wob_ref):
    for x_ref, w_ref, b_ref, o_ref in (
        (x1_ref, w1_ref, b1_ref, o1_ref),
        (x2_ref, w2_ref, b2_ref, o2_ref),
        (x3_ref, w3_ref, b3_ref, o3_ref),
    ):
        xb = x_ref[...].astype(jnp.bfloat16)
        acc = jnp.dot(xb, w_ref[...], preferred_element_type=jnp.float32)
        o_ref[...] = (acc + b_ref[...]).astype(jnp.bfloat16)
    wob_ref[...] = wo_ref[...].astype(jnp.bfloat16)


def _qkv_proj(q2, k2, v2, Wqs, bqs, Wk, bk, Wv, bv, Wo, bm=256):
    M, K = q2.shape
    N = Wqs.shape[1]
    g = M // bm
    ws = N // g
    bf = jnp.bfloat16
    blk_x = pl.BlockSpec((bm, K), lambda i: (i, 0))
    blk_w = pl.BlockSpec((K, N), lambda i: (0, 0))
    blk_b = pl.BlockSpec((1, N), lambda i: (0, 0))
    blk_ws = pl.BlockSpec((ws, N), lambda i: (i, 0))
    blk_o = pl.BlockSpec((bm, N), lambda i: (i, 0))
    out = jax.ShapeDtypeStruct((M, N), bf)
    return pl.pallas_call(
        _qkv_kernel,
        grid=(g,),
        in_specs=[blk_x, blk_x, blk_x, blk_w, blk_w, blk_w,
                  blk_b, blk_b, blk_b, blk_ws],
        out_specs=[blk_o, blk_o, blk_o, blk_ws],
        out_shape=[out, out, out, jax.ShapeDtypeStruct((K, N), bf)],
    )(q2, k2, v2, Wqs, Wk, Wv,
      bqs.reshape(1, N), bk.reshape(1, N), bv.reshape(1, N), Wo)


def _attn_out_kernel(q_ref, k_ref, v_ref, wo_ref, bo_ref, o_ref, ctx_ref):
    D = q_ref.shape[1]
    S = k_ref.shape[0]
    DK = D // H
    C = 256
    nc = S // C
    tokens = []
    for h in range(H):
        lo, hi = h * DK, (h + 1) * DK
        qh = q_ref[:, lo:hi]
        if h >= 3:
            qh = (qh.astype(jnp.float32)
                  + jnp.minimum(tokens[h - 3], 0.0)).astype(jnp.bfloat16)
        m = l = acc = None
        for j in range(nc):
            kj = k_ref[j * C:(j + 1) * C, lo:hi]
            s = jax.lax.dot_general(
                qh, kj, (((1,), (1,)), ((), ())),
                preferred_element_type=jnp.float32,
            )
            mj = jnp.max(s, axis=-1, keepdims=True)
            mnew = mj if m is None else jnp.maximum(m, mj)
            p = jnp.exp2(s - mnew)
            lj = jnp.sum(p, axis=-1, keepdims=True)
            vj = v_ref[j * C:(j + 1) * C, lo:hi]
            dj = jnp.dot(p.astype(jnp.bfloat16), vj,
                         preferred_element_type=jnp.float32)
            if m is None:
                l, acc = lj, dj
            else:
                alpha = jnp.exp2(m - mnew)
                l = l * alpha + lj
                acc = acc * alpha + dj
            m = mnew
        ctx_ref[:, lo:hi] = (acc / l).astype(jnp.bfloat16)
        tokens.append(l)
    o_ref[...] = (
        jnp.dot(ctx_ref[...], wo_ref[...], preferred_element_type=jnp.float32)
        + bo_ref[...]
    )


def _attn_out(qp, kp, vp, Wob, bo, B=2, bq=512):
    from jax.experimental.pallas import tpu as pltpu

    BS, D = qp.shape
    S = BS // B
    nq = S // bq
    outs = []
    for b in range(B):
        outs.append(pl.pallas_call(
            _attn_out_kernel,
            grid=(nq,),
            in_specs=[
                pl.BlockSpec((bq, D), lambda i, b=b: (b * nq + i, 0)),
                pl.BlockSpec((S, D), lambda i, b=b: (b, 0)),
                pl.BlockSpec((S, D), lambda i, b=b: (b, 0)),
                pl.BlockSpec((D, D), lambda i: (0, 0)),
                pl.BlockSpec((1, D), lambda i: (0, 0)),
            ],
            out_specs=pl.BlockSpec((bq, D), lambda i: (i, 0)),
            out_shape=jax.ShapeDtypeStruct((S, D), jnp.float32),
            scratch_shapes=[pltpu.VMEM((bq, D), jnp.bfloat16)],
        )(qp, kp, vp, Wob, bo.reshape(1, D)))
    return jnp.stack(outs)


@jax.jit
def kernel(query, key, value, Wq, bq, Wk, bk, Wv, bv, Wo, bo):
    B, S, D = query.shape
    bf = jnp.bfloat16
    scale = math.log2(math.e) / math.sqrt(D // H)

    Wqs = (Wq * scale).astype(bf)
    Wkb = Wk.astype(bf)
    Wvb = Wv.astype(bf)
    qp, kp, vp, Wob = _qkv_proj(
        query.reshape(B * S, D),
        key.reshape(B * S, D),
        value.reshape(B * S, D),
        Wqs, bq * scale,
        Wkb, bk,
        Wvb, bv,
        Wo,
    )

    out = _attn_out(qp, kp, vp, Wob, bo, B=B)
    return out
